# Initial kernel scaffold; baseline (speedup 1.0000x reference)
import jax
import jax.numpy as jnp
from jax.experimental import pallas as pl


def kernel(score, target):
    raise NotImplementedError("write your pallas kernel here")


# speedup vs baseline: 242.9623x; 242.9623x over previous
import math

import jax
import jax.numpy as jnp
from jax import lax
from jax.experimental import pallas as pl
from jax.experimental.pallas import tpu as pltpu

_THRESH = 0.7
_C07 = math.log(_THRESH / (1.0 - _THRESH))
_RADIUS = 7

_INTERPRET = False


def _per_pixel(score_ref, target_ref):
    t = target_ref[0]
    s0 = score_ref[0, 0]
    s1 = score_ref[0, 1]
    tb = t == 1
    d = jnp.where(tb, s1 - s0, s0 - s1)
    l = jnp.where(tb, -s1, -s0)
    return t, tb, d, l


def _pack_row_scalars(vals):
    row = lax.broadcasted_iota(jnp.int32, (8, 128), 0)
    acc = jnp.zeros((8, 128), jnp.float32)
    for i, v in enumerate(vals):
        acc = jnp.where(row == i, v, acc)
    return acc


def _main_body(score_ref, target_ref, out_ref):
    b = pl.program_id(0)
    t, _, d, l = _per_pixel(score_ref, target_ref)
    H, W = t.shape

    r = lax.broadcasted_iota(jnp.int32, (H, H), 0)
    c = lax.broadcasted_iota(jnp.int32, (H, H), 1)
    band_h = (jnp.abs(r - c) <= _RADIUS).astype(jnp.bfloat16)
    rw = lax.broadcasted_iota(jnp.int32, (W, W), 0)
    cw = lax.broadcasted_iota(jnp.int32, (W, W), 1)
    band_w = (jnp.abs(rw - cw) <= _RADIUS).astype(jnp.bfloat16)

    tf = t.astype(jnp.bfloat16)
    srow = lax.dot_general(band_h, tf, (((1,), (0,)), ((), ())),
                           preferred_element_type=jnp.float32)
    sbox = lax.dot_general(srow.astype(jnp.bfloat16), band_w,
                           (((1,), (0,)), ((), ())),
                           preferred_element_type=jnp.float32)

    ri = lax.broadcasted_iota(jnp.int32, (H, W), 0)
    ci = lax.broadcasted_iota(jnp.int32, (H, W), 1)
    cnt_r = jnp.minimum(ri, _RADIUS) + jnp.minimum(H - 1 - ri, _RADIUS) + 1
    cnt_c = jnp.minimum(ci, _RADIUS) + jnp.minimum(W - 1 - ci, _RADIUS) + 1
    nwin = (cnt_r * cnt_c).astype(jnp.float32)
    boundary = (sbox > 0.0) & (sbox < nwin)

    lt = d < _C07
    le = d <= _C07
    sum_lt = jnp.sum(jnp.where(lt, l, 0.0))
    cnt_lt = jnp.sum(lt.astype(jnp.float32))
    cnt_le = jnp.sum(le.astype(jnp.float32))
    edge_sum = jnp.sum(jnp.where(boundary, l, 0.0))
    edge_cnt = jnp.sum(boundary.astype(jnp.float32))

    acc = _pack_row_scalars([sum_lt, cnt_lt, cnt_le, edge_sum, edge_cnt])

    @pl.when(b == 0)
    def _():
        out_ref[...] = acc

    @pl.when(b != 0)
    def _():
        out_ref[...] = out_ref[...] + acc


def _ikey(d):
    bits = lax.bitcast_convert_type(d, jnp.int32)
    return jnp.where(bits >= 0, bits,
                     jnp.bitwise_xor(jnp.bitwise_not(bits), jnp.int32(-(2 ** 31))))


def _rare_body(kplus1, nb, score_ref, target_ref, out_ref, st_ref):
    i = pl.program_id(0)
    b = pl.program_id(1)

    @pl.when((i == 0) & (b == 0))
    def _():
        st_ref[0] = jnp.int32(-(2 ** 31))
        st_ref[1] = jnp.int32(2 ** 31 - 1)

    @pl.when((i < 32) & (b == 0))
    def _():
        lo = st_ref[0]
        hi = st_ref[1]
        st_ref[2] = (lo >> 1) + (hi >> 1) + (lo & hi & 1)
        st_ref[3] = jnp.int32(0)

    _, tb, d, l = _per_pixel(score_ref, target_ref)
    key = _ikey(d)

    @pl.when(i < 32)
    def _():
        mid = st_ref[2]
        st_ref[3] = st_ref[3] + jnp.sum((key <= mid).astype(jnp.int32))

    @pl.when((i < 32) & (b == nb - 1))
    def _():
        take_hi = st_ref[3] >= kplus1
        lo = st_ref[0]
        hi = st_ref[1]
        mid = st_ref[2]
        st_ref[0] = jnp.where(take_hi, lo, mid + 1)
        st_ref[1] = jnp.where(take_hi, mid, hi)

    @pl.when(i == 32)
    def _():
        kstar = st_ref[0]
        keep = key < kstar
        ssum = jnp.sum(jnp.where(keep, l, 0.0))
        scnt = jnp.sum(keep.astype(jnp.float32))
        acc = _pack_row_scalars([ssum, scnt])

        @pl.when(b == 0)
        def _():
            out_ref[...] = acc

        @pl.when(b != 0)
        def _():
            out_ref[...] = out_ref[...] + acc


def _rare_ohem(score, target, kplus1):
    B, _, H, W = score.shape
    out = pl.pallas_call(
        lambda sr, tr, orf, st: _rare_body(kplus1, B, sr, tr, orf, st),
        grid=(33, B),
        in_specs=[
            pl.BlockSpec((1, 2, H, W), lambda i, b: (b, 0, 0, 0)),
            pl.BlockSpec((1, H, W), lambda i, b: (b, 0, 0)),
        ],
        out_specs=pl.BlockSpec((8, 128), lambda i, b: (0, 0)),
        out_shape=jax.ShapeDtypeStruct((8, 128), jnp.float32),
        scratch_shapes=[pltpu.SMEM((4,), jnp.int32)],
        compiler_params=pltpu.CompilerParams(
            dimension_semantics=("arbitrary", "arbitrary")),
        interpret=_INTERPRET,
    )(score, target)
    ssum = out[0, 0]
    scnt = out[1, 0]
    return ssum / jnp.maximum(scnt, 1.0)


def kernel(score, target):
    B, C, H, W = score.shape
    target = target.astype(jnp.int32)
    min_kept = int(0.7 * H * W)
    k = min(min_kept, B * H * W - 1)

    out = pl.pallas_call(
        _main_body,
        grid=(B,),
        in_specs=[
            pl.BlockSpec((1, C, H, W), lambda b: (b, 0, 0, 0)),
            pl.BlockSpec((1, H, W), lambda b: (b, 0, 0)),
        ],
        out_specs=pl.BlockSpec((8, 128), lambda b: (0, 0)),
        out_shape=jax.ShapeDtypeStruct((8, 128), jnp.float32),
        compiler_params=pltpu.CompilerParams(
            dimension_semantics=("arbitrary",)),
        interpret=_INTERPRET,
    )(score, target)

    sum_lt = out[0, 0]
    cnt_lt = out[1, 0]
    cnt_le = out[2, 0]
    edge_sum = out[3, 0]
    edge_cnt = out[4, 0]

    common = cnt_le >= jnp.float32(k + 1)
    ohem = lax.cond(
        common,
        lambda: sum_lt / jnp.maximum(cnt_lt, 1.0),
        lambda: _rare_ohem(score, target, k + 1),
    )
    edge = edge_sum / jnp.maximum(edge_cnt, 1.0)
    return ohem + 0.5 * edge
